# Initial kernel scaffold; baseline (speedup 1.0000x reference)
import jax
import jax.numpy as jnp
from jax.experimental import pallas as pl


def kernel(features, idx, labels, memory, W, b):
    raise NotImplementedError("write your pallas kernel here")


# speedup vs baseline: 6.6180x; 6.6180x over previous
import functools

import jax
import jax.numpy as jnp
from jax import lax
from jax.experimental import pallas as pl
from jax.experimental.pallas import tpu as pltpu
from jax.experimental.pallas import tpu_sc as plsc

NW = 32
CHUNK = 128
D = 16


def _proj_body(x_ref, w_ref, b_ref, o_ref):
    o_ref[...] = jnp.maximum(
        jnp.dot(x_ref[...], w_ref[...], preferred_element_type=jnp.float32)
        + b_ref[...], 0.0)


def _tc_proj(features, W, b, block_rows):
    n = features.shape[0]
    grid = n // block_rows
    return pl.pallas_call(
        _proj_body,
        grid=(grid,),
        in_specs=[
            pl.BlockSpec((block_rows, features.shape[1]), lambda i: (i, 0)),
            pl.BlockSpec((W.shape[0], W.shape[1]), lambda i: (0, 0)),
            pl.BlockSpec((1, W.shape[1]), lambda i: (0, 0)),
        ],
        out_specs=pl.BlockSpec((block_rows, W.shape[1]), lambda i: (i, 0)),
        out_shape=jax.ShapeDtypeStruct((n, W.shape[1]), jnp.float32),
    )(features, W, b.reshape(1, -1))


def _make_sc_scatter(K, MP):
    mesh = plsc.VectorSubcoreMesh(core_axis_name="c", subcore_axis_name="s")

    @functools.partial(
        pl.kernel,
        out_type=[
            jax.ShapeDtypeStruct((2, MP, D), jnp.float32),
            jax.ShapeDtypeStruct((2, MP, D), jnp.float32),
        ],
        mesh=mesh,
        scratch_types=[
            pltpu.VMEM((K, CHUNK), jnp.int32),
            pltpu.VMEM((K, CHUNK), jnp.int32),
            pltpu.VMEM((CHUNK, D), jnp.float32),
            pltpu.VMEM((CHUNK, D), jnp.float32),
            pltpu.VMEM_SHARED((MP, D), jnp.float32),
            pltpu.VMEM_SHARED((MP, D), jnp.float32),
            pltpu.SemaphoreType.DMA,
        ],
        compiler_params=pltpu.CompilerParams(use_tc_tiling_on_sc=False),
    )
    def sc_scatter(proj_hbm, idx_hbm, lab_hbm, ones_hbm, zeros_hbm,
                   parts_hbm, cnts_hbm,
                   idx_v, lab_v, rows_v, ones_v, bank_sh, cnt_sh, sem):
        c = lax.axis_index("c")
        s = lax.axis_index("s")
        wid = c * 16 + s

        @pl.when(s == 0)
        def _init():
            pltpu.sync_copy(zeros_hbm, bank_sh)
            pltpu.sync_copy(zeros_hbm, cnt_sh)

        pltpu.sync_copy(idx_hbm.at[wid], idx_v)
        pltpu.sync_copy(lab_hbm.at[wid], lab_v)
        pltpu.sync_copy(ones_hbm, ones_v)
        plsc.subcore_barrier()

        def body(j, carry):
            pltpu.async_copy(proj_hbm.at[idx_v.at[j]], rows_v, sem).wait()
            pltpu.sync_copy(rows_v, bank_sh.at[lab_v.at[j]], add=True)
            pltpu.sync_copy(ones_v, cnt_sh.at[lab_v.at[j]], add=True)
            return carry

        lax.fori_loop(0, K, body, 0)
        plsc.subcore_barrier()

        @pl.when(s == 0)
        def _flush():
            pltpu.sync_copy(bank_sh, parts_hbm.at[c])
            pltpu.sync_copy(cnt_sh, cnts_hbm.at[c])

    return sc_scatter


def _ep_body(mem_ref, parts_ref, cnts_ref, o_ref):
    ssum = parts_ref[0] + parts_ref[1]
    cnt = cnts_ref[0, :, 0:1] + cnts_ref[1, :, 0:1]
    o_ref[...] = (mem_ref[...] + ssum) / (cnt + 1e-8)


def _tc_epilogue(memory, parts, cnts):
    M = memory.shape[0]
    return pl.pallas_call(
        _ep_body,
        in_specs=[
            pl.BlockSpec((M, D), lambda: (0, 0)),
            pl.BlockSpec((2, M, D), lambda: (0, 0, 0)),
            pl.BlockSpec((2, M, D), lambda: (0, 0, 0)),
        ],
        out_specs=pl.BlockSpec((M, D), lambda: (0, 0)),
        out_shape=jax.ShapeDtypeStruct((M, D), jnp.float32),
    )(memory, parts, cnts)


def kernel(features, idx, labels, memory, W, b):
    B = idx.shape[0]
    M = memory.shape[0]

    per_tile = -(-B // (NW * CHUNK)) * CHUNK
    K = per_tile // CHUNK
    b_pad = NW * per_tile - B
    mp = -(-(M + 1) // 8) * 8

    idx_p = jnp.concatenate(
        [idx, jnp.zeros((b_pad,), jnp.int32)]).reshape(NW, K, CHUNK)
    lab_p = jnp.concatenate(
        [labels, jnp.full((b_pad,), M, jnp.int32)]).reshape(NW, K, CHUNK)
    ones_pat = jnp.zeros((CHUNK, D), jnp.float32).at[:, 0].set(1.0)
    zeros_pat = jnp.zeros((mp, D), jnp.float32)

    proj = _tc_proj(features, W, b, block_rows=8000)
    parts, cnts = _make_sc_scatter(K, mp)(proj, idx_p, lab_p, ones_pat,
                                          zeros_pat)
    return _tc_epilogue(memory, parts[:, :M], cnts[:, :M])
